# baseline (speedup 1.0000x reference)
import functools

import jax
import jax.numpy as jnp
from jax import lax
from jax.experimental import pallas as pl
from jax.experimental.pallas import tpu as pltpu
from jax.experimental.pallas import tpu_sc as plsc

WALK = 10
E = 128
B = 16384
NC = 2
NS = 16
NW = NC * NS
C = 128
P = 2
K = 2
BUF = P * C

IN_CH = B // (NW * C)
POS_CH = B * WALK // (NW * C)
NEG_CH = B * WALK * 5 // (NW * C)


def _emb_body(in_idx, pos_idx, neg_idx, w_in, w_out, o_in, o_pos, o_neg,
              in_v, pos_v, neg_v, b0, b1, g0, g1, s0, s1):
    bufs = (b0, b1)
    gsems = (g0, g1)
    ssems = (s0, s1)
    wid = lax.axis_index("s") * NC + lax.axis_index("c")

    pltpu.sync_copy(in_idx.at[wid], in_v)
    pltpu.sync_copy(pos_idx.at[wid], pos_v)
    pltpu.sync_copy(neg_idx.at[wid], neg_v)

    def drain_scatter(b, out):
        pltpu.make_async_copy(bufs[b], out.at[pl.ds(0, BUF)], ssems[b]).wait()

    in_base = wid * (B // NW)
    gh = []
    for b in range(K):
        for p in range(P):
            j = b * P + p
            gh.append(pltpu.async_copy(
                w_in.at[in_v.at[j]], bufs[b].at[pl.ds(p * C, C)], gsems[b]))
    for b in range(K):
        for p in range(P):
            gh[b * P + p].wait()
        for k in range(WALK):
            pltpu.async_copy(
                bufs[b], o_in.at[pl.ds(k * B + in_base + b * BUF, BUF)],
                ssems[b])
    for b in range(K):
        for _ in range(WALK):
            drain_scatter(b, o_in)

    def run_phase(idx_v, out, nch, base_row):
        ngrp = nch // (K * P)

        def group(i, carry):
            gh = []
            for b in range(K):
                @pl.when(i != 0)
                def _(b=b):
                    drain_scatter(b, out)
                for p in range(P):
                    ch = (i * K + b) * P + p
                    gh.append(pltpu.async_copy(
                        w_out.at[idx_v.at[ch]],
                        bufs[b].at[pl.ds(p * C, C)], gsems[b]))
            for b in range(K):
                for p in range(P):
                    gh[b * P + p].wait()
                row0 = base_row + (i * K + b) * BUF
                pltpu.async_copy(bufs[b], out.at[pl.ds(row0, BUF)], ssems[b])
            return carry

        lax.fori_loop(0, ngrp, group, 0)
        for b in range(K):
            drain_scatter(b, out)

    run_phase(pos_v, o_pos, POS_CH, wid * POS_CH * C)
    run_phase(neg_v, o_neg, NEG_CH, wid * NEG_CH * C)


_emb = functools.partial(
    pl.kernel,
    mesh=plsc.VectorSubcoreMesh(core_axis_name="c", subcore_axis_name="s"),
    out_type=(
        jax.ShapeDtypeStruct((B * WALK, E), jnp.float32),
        jax.ShapeDtypeStruct((B * WALK, E), jnp.float32),
        jax.ShapeDtypeStruct((B * WALK * 5, E), jnp.float32),
    ),
    scratch_types=[
        pltpu.VMEM((IN_CH, C), jnp.int32),
        pltpu.VMEM((POS_CH, C), jnp.int32),
        pltpu.VMEM((NEG_CH, C), jnp.int32),
    ] + [pltpu.VMEM((BUF, E), jnp.float32) for _ in range(K)]
      + [pltpu.SemaphoreType.DMA for _ in range(2 * K)],
)(_emb_body)


def kernel(input_labels, pos_labels, neg_labels, W_in, W_out):
    in_idx = input_labels.reshape(NW, IN_CH, C).astype(jnp.int32)
    pos_idx = pos_labels.reshape(NW, POS_CH, C).astype(jnp.int32)
    neg_idx = neg_labels.reshape(NW, NEG_CH, C).astype(jnp.int32)
    return _emb(in_idx, pos_idx, neg_idx, W_in, W_out)


# speedup vs baseline: 1.0111x; 1.0111x over previous
import functools

import jax
import jax.numpy as jnp
from jax import lax
from jax.experimental import pallas as pl
from jax.experimental.pallas import tpu as pltpu
from jax.experimental.pallas import tpu_sc as plsc

WALK = 10
E = 128
B = 16384
NC = 2
NS = 16
NW = NC * NS
C = 128
P = 1
K = 4
BUF = P * C

IN_CH = B // (NW * C)
POS_CH = B * WALK // (NW * C)
NEG_CH = B * WALK * 5 // (NW * C)


def _emb_body(in_idx, pos_idx, neg_idx, w_in, w_out, o_in, o_pos, o_neg,
              in_v, pos_v, neg_v, b0, b1, b2, b3, g0, g1, g2, g3, s0, s1, s2, s3):
    bufs = (b0, b1, b2, b3)
    gsems = (g0, g1, g2, g3)
    ssems = (s0, s1, s2, s3)
    wid = lax.axis_index("s") * NC + lax.axis_index("c")

    pltpu.sync_copy(in_idx.at[wid], in_v)
    pltpu.sync_copy(pos_idx.at[wid], pos_v)
    pltpu.sync_copy(neg_idx.at[wid], neg_v)

    def drain_scatter(b, out):
        pltpu.make_async_copy(bufs[b], out.at[pl.ds(0, BUF)], ssems[b]).wait()

    in_base = wid * (B // NW)
    gh = []
    for b in range(K):
        for p in range(P):
            j = b * P + p
            gh.append(pltpu.async_copy(
                w_in.at[in_v.at[j]], bufs[b].at[pl.ds(p * C, C)], gsems[b]))
    for b in range(K):
        for p in range(P):
            gh[b * P + p].wait()
        for k in range(WALK):
            pltpu.async_copy(
                bufs[b], o_in.at[pl.ds(k * B + in_base + b * BUF, BUF)],
                ssems[b])
    for b in range(K):
        for _ in range(WALK):
            drain_scatter(b, o_in)

    def run_phase(idx_v, out, nch, base_row):
        ngrp = nch // (K * P)

        def group(i, carry):
            gh = []
            for b in range(K):
                @pl.when(i != 0)
                def _(b=b):
                    drain_scatter(b, out)
                for p in range(P):
                    ch = (i * K + b) * P + p
                    gh.append(pltpu.async_copy(
                        w_out.at[idx_v.at[ch]],
                        bufs[b].at[pl.ds(p * C, C)], gsems[b]))
            for b in range(K):
                for p in range(P):
                    gh[b * P + p].wait()
                row0 = base_row + (i * K + b) * BUF
                pltpu.async_copy(bufs[b], out.at[pl.ds(row0, BUF)], ssems[b])
            return carry

        lax.fori_loop(0, ngrp, group, 0)
        for b in range(K):
            drain_scatter(b, out)

    run_phase(pos_v, o_pos, POS_CH, wid * POS_CH * C)
    run_phase(neg_v, o_neg, NEG_CH, wid * NEG_CH * C)


_emb = functools.partial(
    pl.kernel,
    mesh=plsc.VectorSubcoreMesh(core_axis_name="c", subcore_axis_name="s"),
    out_type=(
        jax.ShapeDtypeStruct((B * WALK, E), jnp.float32),
        jax.ShapeDtypeStruct((B * WALK, E), jnp.float32),
        jax.ShapeDtypeStruct((B * WALK * 5, E), jnp.float32),
    ),
    scratch_types=[
        pltpu.VMEM((IN_CH, C), jnp.int32),
        pltpu.VMEM((POS_CH, C), jnp.int32),
        pltpu.VMEM((NEG_CH, C), jnp.int32),
    ] + [pltpu.VMEM((BUF, E), jnp.float32) for _ in range(K)]
      + [pltpu.SemaphoreType.DMA for _ in range(2 * K)],
)(_emb_body)


def kernel(input_labels, pos_labels, neg_labels, W_in, W_out):
    in_idx = input_labels.reshape(NW, IN_CH, C).astype(jnp.int32)
    pos_idx = pos_labels.reshape(NW, POS_CH, C).astype(jnp.int32)
    neg_idx = neg_labels.reshape(NW, NEG_CH, C).astype(jnp.int32)
    return _emb(in_idx, pos_idx, neg_idx, W_in, W_out)


# speedup vs baseline: 1.8415x; 1.8213x over previous
import functools

import jax
import jax.numpy as jnp
from jax import lax
from jax.experimental import pallas as pl
from jax.experimental.pallas import tpu as pltpu
from jax.experimental.pallas import tpu_sc as plsc

WALK = 10
E = 128
B = 16384
NC = 2
NS = 16
NW = NC * NS
C = 128
P = 1
K = 4
BUF = P * C

IN_CH = B // (NW * C)
POS_CH = B * WALK // (NW * C)
NEG_CH = B * WALK * 5 // (NW * C)


def _emb_body(in_idx, pos_idx, neg_idx, w_in, w_out, o_in, o_pos, o_neg,
              w_out_sh,
              in_v, pos_v, neg_v, b0, b1, b2, b3, g0, g1, g2, g3, s0, s1, s2, s3):
    bufs = (b0, b1, b2, b3)
    gsems = (g0, g1, g2, g3)
    ssems = (s0, s1, s2, s3)
    sid = lax.axis_index("s")
    wid = sid * NC + lax.axis_index("c")

    @pl.when(sid == 0)
    def _():
        pltpu.sync_copy(w_out, w_out_sh)

    pltpu.sync_copy(in_idx.at[wid], in_v)
    pltpu.sync_copy(pos_idx.at[wid], pos_v)
    pltpu.sync_copy(neg_idx.at[wid], neg_v)
    plsc.subcore_barrier()

    def drain_scatter(b, out):
        pltpu.make_async_copy(bufs[b], out.at[pl.ds(0, BUF)], ssems[b]).wait()

    in_base = wid * (B // NW)
    gh = []
    for b in range(K):
        for p in range(P):
            j = b * P + p
            gh.append(pltpu.async_copy(
                w_in.at[in_v.at[j]], bufs[b].at[pl.ds(p * C, C)], gsems[b]))
    for b in range(K):
        for p in range(P):
            gh[b * P + p].wait()
        for k in range(WALK):
            pltpu.async_copy(
                bufs[b], o_in.at[pl.ds(k * B + in_base + b * BUF, BUF)],
                ssems[b])
    for b in range(K):
        for _ in range(WALK):
            drain_scatter(b, o_in)

    def run_phase(idx_v, out, nch, base_row):
        ngrp = nch // (K * P)

        def group(i, carry):
            gh = []
            for b in range(K):
                @pl.when(i != 0)
                def _(b=b):
                    drain_scatter(b, out)
                for p in range(P):
                    ch = (i * K + b) * P + p
                    gh.append(pltpu.async_copy(
                        w_out_sh.at[idx_v.at[ch]],
                        bufs[b].at[pl.ds(p * C, C)], gsems[b]))
            for b in range(K):
                for p in range(P):
                    gh[b * P + p].wait()
                row0 = base_row + (i * K + b) * BUF
                pltpu.async_copy(bufs[b], out.at[pl.ds(row0, BUF)], ssems[b])
            return carry

        lax.fori_loop(0, ngrp, group, 0)
        for b in range(K):
            drain_scatter(b, out)

    run_phase(pos_v, o_pos, POS_CH, wid * POS_CH * C)
    run_phase(neg_v, o_neg, NEG_CH, wid * NEG_CH * C)


_emb = functools.partial(
    pl.kernel,
    mesh=plsc.VectorSubcoreMesh(core_axis_name="c", subcore_axis_name="s"),
    out_type=(
        jax.ShapeDtypeStruct((B * WALK, E), jnp.float32),
        jax.ShapeDtypeStruct((B * WALK, E), jnp.float32),
        jax.ShapeDtypeStruct((B * WALK * 5, E), jnp.float32),
    ),
    scratch_types=[
        pltpu.VMEM_SHARED((2405, E), jnp.float32),
        pltpu.VMEM((IN_CH, C), jnp.int32),
        pltpu.VMEM((POS_CH, C), jnp.int32),
        pltpu.VMEM((NEG_CH, C), jnp.int32),
    ] + [pltpu.VMEM((BUF, E), jnp.float32) for _ in range(K)]
      + [pltpu.SemaphoreType.DMA for _ in range(2 * K)],
)(_emb_body)


def kernel(input_labels, pos_labels, neg_labels, W_in, W_out):
    in_idx = input_labels.reshape(NW, IN_CH, C).astype(jnp.int32)
    pos_idx = pos_labels.reshape(NW, POS_CH, C).astype(jnp.int32)
    neg_idx = neg_labels.reshape(NW, NEG_CH, C).astype(jnp.int32)
    return _emb(in_idx, pos_idx, neg_idx, W_in, W_out)
